# Initial kernel scaffold; baseline (speedup 1.0000x reference)
import functools

import jax
import jax.numpy as jnp
import numpy as np
from jax import lax
from jax.experimental import pallas as pl
from jax.experimental.pallas import tpu as pltpu
from jax.experimental.pallas import tpu_sc as plsc

RES = 32
NV = RES * RES * RES
KNN = 8
NP = 20000
NPAD = 20480
HID = 128
PE_DIM = 32
PCH = 3 * PE_DIM
SCALER = 32.0

BQ = 128
VB = 1024
NE = NV * KNN

_F32 = jnp.float32


def _host_grid_constants():
    ax = np.linspace(0.0, 1.0, RES, dtype=np.float32)
    g = np.stack(np.meshgrid(ax, ax, ax, indexing='ij'), axis=-1)
    grid_flat = g.reshape(-1, 3).astype(np.float32)
    freqs = (2.0 * np.pi) * (2.0 ** np.arange(PE_DIM // 2, dtype=np.float32))
    xf = grid_flat[..., None].astype(np.float32) * freqs
    enc = np.concatenate([np.sin(xf), np.cos(xf)], axis=-1)
    grid_feat = enc.reshape(NV, 3 * PE_DIM).astype(np.float32)
    qaug = np.zeros((NV, 8), dtype=np.float32)
    qaug[:, 0:3] = -grid_flat * SCALER
    qaug[:, 3] = 1.0
    return grid_flat, grid_feat, qaug


_GRID_FLAT, _GRID_FEAT, _QAUG = _host_grid_constants()


def _prep_body(feats_ref, pts_ref, ptsT_ref, w1a_ref, w1c_ref, p_ref, paug_ref):
    p_ref[...] = (
        jnp.dot(feats_ref[...], w1a_ref[...], preferred_element_type=_F32)
        + jnp.dot(pts_ref[...], w1c_ref[...], preferred_element_type=_F32)
    )
    pt = ptsT_ref[...]
    p2 = jnp.sum(pt * pt, axis=0, keepdims=True)
    paug_ref[...] = jnp.concatenate(
        [pt, 0.5 * p2, jnp.zeros((4, NPAD), _F32)], axis=0)


def _prep(feats, pts, ptsT_scaled, w1a, w1c):
    return pl.pallas_call(
        _prep_body,
        out_shape=[
            jax.ShapeDtypeStruct((NP, HID), _F32),
            jax.ShapeDtypeStruct((8, NPAD), _F32),
        ],
    )(feats, pts, ptsT_scaled, w1a, w1c)


def _knn_body(qaug_ref, paug_ref, nbr_ref):
    s = jnp.dot(qaug_ref[...], paug_ref[...], preferred_element_type=_F32)
    iota = lax.broadcasted_iota(jnp.int32, s.shape, 1)
    cols = []
    for _ in range(KNN):
        m = jnp.min(s, axis=1, keepdims=True)
        cand = jnp.where(s == m, iota, jnp.int32(2 ** 30))
        idx = jnp.min(cand, axis=1, keepdims=True)
        cols.append(idx)
        s = jnp.where(iota == idx, jnp.float32(3e38), s)
    nbr_ref[...] = jnp.concatenate(cols, axis=1)


def _knn(qaug, paug):
    return pl.pallas_call(
        _knn_body,
        grid=(NV // BQ,),
        in_specs=[
            pl.BlockSpec((BQ, 8), lambda i: (i, 0)),
            pl.BlockSpec((8, NPAD), lambda i: (0, 0)),
        ],
        out_specs=pl.BlockSpec((BQ, KNN), lambda i: (i, 0)),
        out_shape=jax.ShapeDtypeStruct((NV, KNN), jnp.int32),
    )(qaug, paug)


def _sc_gather(p_table, nbr2d):
    info = plsc.get_sparse_core_info()
    nc, ns = info.num_cores, info.num_subcores
    nw = nc * ns
    rows = nbr2d.shape[0]
    rpw = rows // nw

    @functools.partial(
        pl.kernel,
        out_type=jax.ShapeDtypeStruct((NE, HID), _F32),
        mesh=plsc.VectorSubcoreMesh(core_axis_name="c", subcore_axis_name="s"),
        scratch_types=[
            pltpu.VMEM((rpw, 128), jnp.int32),
            pltpu.VMEM((128, HID), _F32),
            pltpu.SemaphoreType.DMA,
        ],
    )
    def gather_kernel(table_hbm, idx_hbm, out_hbm, idx_v, rows_v, sem):
        wid = lax.axis_index("s") * nc + lax.axis_index("c")
        base = wid * rpw
        pltpu.sync_copy(idx_hbm.at[pl.ds(base, rpw)], idx_v)

        def body(j, carry):
            pltpu.async_copy(table_hbm.at[idx_v.at[j]], rows_v, sem).wait()
            pltpu.sync_copy(rows_v, out_hbm.at[pl.ds((base + j) * 128, 128)])
            return carry

        lax.fori_loop(0, rpw, body, 0)

    return gather_kernel(p_table, nbr2d)


def _mlp_body(gath_ref, gfeat_ref, gneg_ref, w1b_ref, w1c_ref, b1_ref,
              w2_ref, b2_ref, w3_ref, b3_ref, w4_ref, b4_ref, out_ref):
    G = (
        jnp.dot(gfeat_ref[...], w1b_ref[...], preferred_element_type=_F32)
        + jnp.dot(gneg_ref[...], w1c_ref[...], preferred_element_type=_F32)
        + b1_ref[...]
    )
    g3 = gath_ref[...].reshape(VB, KNN, HID)
    hsum = jnp.zeros((VB, HID), _F32)
    for k in range(KNN):
        hsum = hsum + jax.nn.gelu(g3[:, k, :] + G)
    hbar = hsum * (1.0 / KNN)
    red = jnp.dot(hbar, w2_ref[...], preferred_element_type=_F32) + b2_ref[...]
    h2 = jax.nn.gelu(
        jnp.dot(red, w3_ref[...], preferred_element_type=_F32) + b3_ref[...])
    out_ref[...] = (
        jnp.dot(h2, w4_ref[...], preferred_element_type=_F32) + b4_ref[...])


def _mlp(gath, gfeat, gneg, w1b, w1c, b1, w2, b2, w3, b3, w4, b4):
    full = lambda shape: pl.BlockSpec(shape, lambda i: (0, 0))
    return pl.pallas_call(
        _mlp_body,
        grid=(NV // VB,),
        in_specs=[
            pl.BlockSpec((VB * KNN, HID), lambda i: (i, 0)),
            pl.BlockSpec((VB, PCH), lambda i: (i, 0)),
            pl.BlockSpec((VB, 3), lambda i: (i, 0)),
            full((PCH, HID)), full((3, HID)), full((1, HID)),
            full((HID, 64)), full((1, 64)),
            full((64, HID)), full((1, HID)),
            full((HID, 64)), full((1, 64)),
        ],
        out_specs=pl.BlockSpec((VB, 64), lambda i: (i, 0)),
        out_shape=jax.ShapeDtypeStruct((NV, 64), _F32),
    )(gath, gfeat, gneg, w1b, w1c, b1, w2, b2, w3, b3, w4, b4)


def kernel(vertices, features, W1, b1, W2, b2, W3, b3, W4, b4):
    pts = vertices[0]
    feats = features[0]
    w1a, w1b, w1c = W1[:64], W1[64:160], W1[160:]

    pad = jnp.full((NPAD - NP, 3), 1e15, _F32)
    ptsT_scaled = jnp.concatenate([pts * SCALER, pad], axis=0).T

    p_table, paug = _prep(feats, pts, ptsT_scaled, w1a, w1c)
    nbr = _knn(jnp.asarray(_QAUG), paug)
    gath = _sc_gather(p_table, nbr.reshape(NE // 128, 128))
    out = _mlp(
        gath, jnp.asarray(_GRID_FEAT), jnp.asarray(-_GRID_FLAT),
        w1b, w1c, b1.reshape(1, HID),
        W2, b2.reshape(1, 64), W3, b3.reshape(1, HID), W4, b4.reshape(1, 64))
    return out.reshape(1, RES, RES, RES, 64)


# speedup vs baseline: 2.4340x; 2.4340x over previous
import functools

import jax
import jax.numpy as jnp
import numpy as np
from jax import lax
from jax.experimental import pallas as pl
from jax.experimental.pallas import tpu as pltpu
from jax.experimental.pallas import tpu_sc as plsc

RES = 32
NV = RES * RES * RES
KNN = 8
NP = 20000
NPAD = 20480
HID = 128
PE_DIM = 32
PCH = 3 * PE_DIM
SCALER = 32.0

BQ = 128
VB = 1024
NE = NV * KNN

_F32 = jnp.float32


def _host_grid_constants():
    ax = np.linspace(0.0, 1.0, RES, dtype=np.float32)
    g = np.stack(np.meshgrid(ax, ax, ax, indexing='ij'), axis=-1)
    grid_flat = g.reshape(-1, 3).astype(np.float32)
    freqs = (2.0 * np.pi) * (2.0 ** np.arange(PE_DIM // 2, dtype=np.float32))
    xf = grid_flat[..., None].astype(np.float32) * freqs
    enc = np.concatenate([np.sin(xf), np.cos(xf)], axis=-1)
    grid_feat = enc.reshape(NV, 3 * PE_DIM).astype(np.float32)
    q = grid_flat * SCALER
    qbf = np.zeros((NV, 8), dtype=np.float32)
    qbf[:, 0:3] = q
    q2 = (q * q).sum(axis=1, keepdims=True).astype(np.float32)
    return grid_flat, grid_feat, qbf, q2


_GRID_FLAT, _GRID_FEAT, _QBF_F32, _Q2 = _host_grid_constants()


def _prep_body(feats_ref, pts_ref, ptsT_ref, w1a_ref, w1c_ref,
               p_ref, paug_ref, p2_ref):
    p_ref[...] = (
        jnp.dot(feats_ref[...], w1a_ref[...], preferred_element_type=_F32)
        + jnp.dot(pts_ref[...], w1c_ref[...], preferred_element_type=_F32)
    )
    pt = ptsT_ref[...]
    p2_ref[...] = jnp.sum(pt * pt, axis=0, keepdims=True)
    paug_ref[...] = jnp.concatenate(
        [pt, jnp.zeros((5, NPAD), _F32)], axis=0).astype(jnp.bfloat16)


def _prep(feats, pts, ptsT_scaled, w1a, w1c):
    return pl.pallas_call(
        _prep_body,
        out_shape=[
            jax.ShapeDtypeStruct((NP, HID), _F32),
            jax.ShapeDtypeStruct((8, NPAD), jnp.bfloat16),
            jax.ShapeDtypeStruct((1, NPAD), _F32),
        ],
    )(feats, pts, ptsT_scaled, w1a, w1c)


def _knn_body(qbf_ref, q2_ref, paug_ref, p2_ref, nbr_ref):
    qp = jnp.dot(qbf_ref[...], paug_ref[...], preferred_element_type=_F32)
    s = (q2_ref[...] + p2_ref[...]) - 2.0 * qp
    iota = lax.broadcasted_iota(jnp.int32, s.shape, 1)
    cols = []
    for _ in range(KNN):
        m = jnp.min(s, axis=1, keepdims=True)
        cand = jnp.where(s == m, iota, jnp.int32(2 ** 30))
        idx = jnp.min(cand, axis=1, keepdims=True)
        cols.append(idx)
        s = jnp.where(iota == idx, jnp.float32(3e38), s)
    nbr_ref[...] = jnp.concatenate(cols, axis=1)


def _knn(qbf, q2, paug, p2row):
    return pl.pallas_call(
        _knn_body,
        grid=(NV // BQ,),
        in_specs=[
            pl.BlockSpec((BQ, 8), lambda i: (i, 0)),
            pl.BlockSpec((BQ, 1), lambda i: (i, 0)),
            pl.BlockSpec((8, NPAD), lambda i: (0, 0)),
            pl.BlockSpec((1, NPAD), lambda i: (0, 0)),
        ],
        out_specs=pl.BlockSpec((BQ, KNN), lambda i: (i, 0)),
        out_shape=jax.ShapeDtypeStruct((NV, KNN), jnp.int32),
    )(qbf, q2, paug, p2row)


def _sc_gather(p_table, nbr2d):
    info = plsc.get_sparse_core_info()
    nc, ns = info.num_cores, info.num_subcores
    nw = nc * ns
    rows = nbr2d.shape[0]
    rpw = rows // nw

    @functools.partial(
        pl.kernel,
        out_type=jax.ShapeDtypeStruct((NE, HID), _F32),
        mesh=plsc.VectorSubcoreMesh(core_axis_name="c", subcore_axis_name="s"),
        scratch_types=[
            pltpu.VMEM((rpw, 128), jnp.int32),
            pltpu.VMEM((128, HID), _F32),
            pltpu.SemaphoreType.DMA,
        ],
    )
    def gather_kernel(table_hbm, idx_hbm, out_hbm, idx_v, rows_v, sem):
        wid = lax.axis_index("s") * nc + lax.axis_index("c")
        base = wid * rpw
        pltpu.sync_copy(idx_hbm.at[pl.ds(base, rpw)], idx_v)

        def body(j, carry):
            pltpu.async_copy(table_hbm.at[idx_v.at[j]], rows_v, sem).wait()
            pltpu.sync_copy(rows_v, out_hbm.at[pl.ds((base + j) * 128, 128)])
            return carry

        lax.fori_loop(0, rpw, body, 0)

    return gather_kernel(p_table, nbr2d)


def _mlp_body(gath_ref, gfeat_ref, gneg_ref, w1b_ref, w1c_ref, b1_ref,
              w2_ref, b2_ref, w3_ref, b3_ref, w4_ref, b4_ref, out_ref):
    G = (
        jnp.dot(gfeat_ref[...], w1b_ref[...], preferred_element_type=_F32)
        + jnp.dot(gneg_ref[...], w1c_ref[...], preferred_element_type=_F32)
        + b1_ref[...]
    )
    g3 = gath_ref[...].reshape(VB, KNN, HID)
    hsum = jnp.zeros((VB, HID), _F32)
    for k in range(KNN):
        hsum = hsum + jax.nn.gelu(g3[:, k, :] + G)
    hbar = hsum * (1.0 / KNN)
    red = jnp.dot(hbar, w2_ref[...], preferred_element_type=_F32) + b2_ref[...]
    h2 = jax.nn.gelu(
        jnp.dot(red, w3_ref[...], preferred_element_type=_F32) + b3_ref[...])
    out_ref[...] = (
        jnp.dot(h2, w4_ref[...], preferred_element_type=_F32) + b4_ref[...])


def _mlp(gath, gfeat, gneg, w1b, w1c, b1, w2, b2, w3, b3, w4, b4):
    full = lambda shape: pl.BlockSpec(shape, lambda i: (0, 0))
    return pl.pallas_call(
        _mlp_body,
        grid=(NV // VB,),
        in_specs=[
            pl.BlockSpec((VB * KNN, HID), lambda i: (i, 0)),
            pl.BlockSpec((VB, PCH), lambda i: (i, 0)),
            pl.BlockSpec((VB, 3), lambda i: (i, 0)),
            full((PCH, HID)), full((3, HID)), full((1, HID)),
            full((HID, 64)), full((1, 64)),
            full((64, HID)), full((1, HID)),
            full((HID, 64)), full((1, 64)),
        ],
        out_specs=pl.BlockSpec((VB, 64), lambda i: (i, 0)),
        out_shape=jax.ShapeDtypeStruct((NV, 64), _F32),
    )(gath, gfeat, gneg, w1b, w1c, b1, w2, b2, w3, b3, w4, b4)


def kernel(vertices, features, W1, b1, W2, b2, W3, b3, W4, b4):
    pts = vertices[0]
    feats = features[0]
    w1a, w1b, w1c = W1[:64], W1[64:160], W1[160:]

    pad = jnp.full((NPAD - NP, 3), 1e15, _F32)
    ptsT_scaled = jnp.concatenate([pts * SCALER, pad], axis=0).T

    p_table, paug, p2row = _prep(feats, pts, ptsT_scaled, w1a, w1c)
    qbf = jnp.asarray(_QBF_F32).astype(jnp.bfloat16)
    nbr = _knn(qbf, jnp.asarray(_Q2), paug, p2row)
    gath = _sc_gather(p_table, nbr.reshape(NE // 128, 128))
    out = _mlp(
        gath, jnp.asarray(_GRID_FEAT), jnp.asarray(-_GRID_FLAT),
        w1b, w1c, b1.reshape(1, HID),
        W2, b2.reshape(1, 64), W3, b3.reshape(1, HID), W4, b4.reshape(1, 64))
    return out.reshape(1, RES, RES, RES, 64)


# speedup vs baseline: 3.0472x; 1.2519x over previous
import functools

import jax
import jax.numpy as jnp
import numpy as np
from jax import lax
from jax.experimental import pallas as pl
from jax.experimental.pallas import tpu as pltpu
from jax.experimental.pallas import tpu_sc as plsc

RES = 32
NV = RES * RES * RES
KNN = 8
NP = 20000
NPAD = 20480
HID = 128
PE_DIM = 32
PCH = 3 * PE_DIM
SCALER = 32.0

BQ = 64
VB = 1024
NE = NV * KNN

_F32 = jnp.float32


def _host_grid_constants():
    ax = np.linspace(0.0, 1.0, RES, dtype=np.float32)
    g = np.stack(np.meshgrid(ax, ax, ax, indexing='ij'), axis=-1)
    grid_flat = g.reshape(-1, 3).astype(np.float32)
    freqs = (2.0 * np.pi) * (2.0 ** np.arange(PE_DIM // 2, dtype=np.float32))
    xf = grid_flat[..., None].astype(np.float32) * freqs
    enc = np.concatenate([np.sin(xf), np.cos(xf)], axis=-1)
    grid_feat = enc.reshape(NV, 3 * PE_DIM).astype(np.float32)
    q = grid_flat * SCALER
    qbf = np.zeros((NV, 8), dtype=np.float32)
    qbf[:, 0:3] = q
    q2 = (q * q).sum(axis=1, keepdims=True).astype(np.float32)
    return grid_flat, grid_feat, qbf, q2


_GRID_FLAT, _GRID_FEAT, _QBF_F32, _Q2 = _host_grid_constants()


def _prep_body(feats_ref, pts_ref, ptsT_ref, w1a_ref, w1c_ref,
               p_ref, paug_ref, p2_ref):
    p_ref[...] = (
        jnp.dot(feats_ref[...], w1a_ref[...], preferred_element_type=_F32)
        + jnp.dot(pts_ref[...], w1c_ref[...], preferred_element_type=_F32)
    )
    pt = ptsT_ref[...]
    p2_ref[...] = jnp.sum(pt * pt, axis=0, keepdims=True)
    paug_ref[...] = jnp.concatenate(
        [pt, jnp.zeros((5, NPAD), _F32)], axis=0).astype(jnp.bfloat16)


def _prep(feats, pts, ptsT_scaled, w1a, w1c):
    return pl.pallas_call(
        _prep_body,
        out_shape=[
            jax.ShapeDtypeStruct((NP, HID), _F32),
            jax.ShapeDtypeStruct((8, NPAD), jnp.bfloat16),
            jax.ShapeDtypeStruct((1, NPAD), _F32),
        ],
    )(feats, pts, ptsT_scaled, w1a, w1c)


def _knn_body(qbf_ref, q2_ref, paug_ref, p2_ref, nbr_ref):
    qp = jnp.dot(qbf_ref[...], paug_ref[...], preferred_element_type=_F32)
    s = (q2_ref[...] + p2_ref[...]) - 2.0 * qp
    iota_f = lax.broadcasted_iota(jnp.int32, s.shape, 1).astype(_F32)
    big = jnp.float32(3e38)
    cols = []
    for k in range(KNN):
        m = jnp.min(s, axis=1, keepdims=True)
        hit = s == m
        idxf = jnp.min(jnp.where(hit, iota_f, big), axis=1, keepdims=True)
        cols.append(idxf)
        if k < KNN - 1:
            s = jnp.where(hit, big, s)
    nbr_ref[...] = jnp.concatenate(cols, axis=1).astype(jnp.int32)


def _knn(qbf, q2, paug, p2row):
    return pl.pallas_call(
        _knn_body,
        grid=(NV // BQ,),
        in_specs=[
            pl.BlockSpec((BQ, 8), lambda i: (i, 0)),
            pl.BlockSpec((BQ, 1), lambda i: (i, 0)),
            pl.BlockSpec((8, NPAD), lambda i: (0, 0)),
            pl.BlockSpec((1, NPAD), lambda i: (0, 0)),
        ],
        out_specs=pl.BlockSpec((BQ, KNN), lambda i: (i, 0)),
        out_shape=jax.ShapeDtypeStruct((NV, KNN), jnp.int32),
    )(qbf, q2, paug, p2row)


def _sc_gather(p_table, nbr2d):
    info = plsc.get_sparse_core_info()
    nc, ns = info.num_cores, info.num_subcores
    nw = nc * ns
    rows = nbr2d.shape[0]
    rpw = rows // nw

    @functools.partial(
        pl.kernel,
        out_type=jax.ShapeDtypeStruct((NE, HID), _F32),
        mesh=plsc.VectorSubcoreMesh(core_axis_name="c", subcore_axis_name="s"),
        scratch_types=[
            pltpu.VMEM((rpw, 128), jnp.int32),
            pltpu.VMEM((128, HID), _F32),
            pltpu.SemaphoreType.DMA,
        ],
    )
    def gather_kernel(table_hbm, idx_hbm, out_hbm, idx_v, rows_v, sem):
        wid = lax.axis_index("s") * nc + lax.axis_index("c")
        base = wid * rpw
        pltpu.sync_copy(idx_hbm.at[pl.ds(base, rpw)], idx_v)

        def body(j, carry):
            pltpu.async_copy(table_hbm.at[idx_v.at[j]], rows_v, sem).wait()
            pltpu.sync_copy(rows_v, out_hbm.at[pl.ds((base + j) * 128, 128)])
            return carry

        lax.fori_loop(0, rpw, body, 0)

    return gather_kernel(p_table, nbr2d)


def _mlp_body(gath_ref, gfeat_ref, gneg_ref, w1b_ref, w1c_ref, b1_ref,
              w2_ref, b2_ref, w3_ref, b3_ref, w4_ref, b4_ref, out_ref):
    G = (
        jnp.dot(gfeat_ref[...], w1b_ref[...], preferred_element_type=_F32)
        + jnp.dot(gneg_ref[...], w1c_ref[...], preferred_element_type=_F32)
        + b1_ref[...]
    )
    hsum = jnp.zeros((VB, HID), _F32)
    for k in range(KNN):
        hsum = hsum + jax.nn.gelu(gath_ref[k] + G)
    hbar = hsum * (1.0 / KNN)
    red = jnp.dot(hbar, w2_ref[...], preferred_element_type=_F32) + b2_ref[...]
    h2 = jax.nn.gelu(
        jnp.dot(red, w3_ref[...], preferred_element_type=_F32) + b3_ref[...])
    out_ref[...] = (
        jnp.dot(h2, w4_ref[...], preferred_element_type=_F32) + b4_ref[...])


def _mlp(gath, gfeat, gneg, w1b, w1c, b1, w2, b2, w3, b3, w4, b4):
    full = lambda shape: pl.BlockSpec(shape, lambda i: (0, 0))
    return pl.pallas_call(
        _mlp_body,
        grid=(NV // VB,),
        in_specs=[
            pl.BlockSpec((KNN, VB, HID), lambda i: (0, i, 0)),
            pl.BlockSpec((VB, PCH), lambda i: (i, 0)),
            pl.BlockSpec((VB, 3), lambda i: (i, 0)),
            full((PCH, HID)), full((3, HID)), full((1, HID)),
            full((HID, 64)), full((1, 64)),
            full((64, HID)), full((1, HID)),
            full((HID, 64)), full((1, 64)),
        ],
        out_specs=pl.BlockSpec((VB, 64), lambda i: (i, 0)),
        out_shape=jax.ShapeDtypeStruct((NV, 64), _F32),
    )(gath, gfeat, gneg, w1b, w1c, b1, w2, b2, w3, b3, w4, b4)


def kernel(vertices, features, W1, b1, W2, b2, W3, b3, W4, b4):
    pts = vertices[0]
    feats = features[0]
    w1a, w1b, w1c = W1[:64], W1[64:160], W1[160:]

    pad = jnp.full((NPAD - NP, 3), 1e15, _F32)
    ptsT_scaled = jnp.concatenate([pts * SCALER, pad], axis=0).T

    p_table, paug, p2row = _prep(feats, pts, ptsT_scaled, w1a, w1c)
    qbf = jnp.asarray(_QBF_F32).astype(jnp.bfloat16)
    nbr = _knn(qbf, jnp.asarray(_Q2), paug, p2row)
    gath = _sc_gather(p_table, nbr.T.reshape(NE // 128, 128))
    out = _mlp(
        gath.reshape(KNN, NV, HID),
        jnp.asarray(_GRID_FEAT), jnp.asarray(-_GRID_FLAT),
        w1b, w1c, b1.reshape(1, HID),
        W2, b2.reshape(1, 64), W3, b3.reshape(1, HID), W4, b4.reshape(1, 64))
    return out.reshape(1, RES, RES, RES, 64)


# speedup vs baseline: 3.3162x; 1.0883x over previous
import functools

import jax
import jax.numpy as jnp
import numpy as np
from jax import lax
from jax.experimental import pallas as pl
from jax.experimental.pallas import tpu as pltpu
from jax.experimental.pallas import tpu_sc as plsc

RES = 32
NV = RES * RES * RES
KNN = 8
NP = 20000
NPAD = 20480
HID = 128
PE_DIM = 32
PCH = 3 * PE_DIM
SCALER = 32.0

BQ = 64
VB = 1024
NE = NV * KNN

_F32 = jnp.float32


def _host_grid_constants():
    ax = np.linspace(0.0, 1.0, RES, dtype=np.float32)
    g = np.stack(np.meshgrid(ax, ax, ax, indexing='ij'), axis=-1)
    grid_flat = g.reshape(-1, 3).astype(np.float32)
    freqs = (2.0 * np.pi) * (2.0 ** np.arange(PE_DIM // 2, dtype=np.float32))
    xf = grid_flat[..., None].astype(np.float32) * freqs
    enc = np.concatenate([np.sin(xf), np.cos(xf)], axis=-1)
    grid_feat = enc.reshape(NV, 3 * PE_DIM).astype(np.float32)
    q = grid_flat * SCALER
    qbf = np.zeros((NV, 8), dtype=np.float32)
    qbf[:, 0:3] = q
    q2 = (q * q).sum(axis=1, keepdims=True).astype(np.float32)
    return grid_flat, grid_feat, qbf, q2


_GRID_FLAT, _GRID_FEAT, _QBF_F32, _Q2 = _host_grid_constants()


def _prep_body(feats_ref, pts_ref, ptsT_ref, w1a_ref, w1c_ref,
               p_ref, paug_ref, p2_ref):
    p_ref[...] = (
        jnp.dot(feats_ref[...], w1a_ref[...], preferred_element_type=_F32)
        + jnp.dot(pts_ref[...], w1c_ref[...], preferred_element_type=_F32)
    )
    pt = ptsT_ref[...]
    p2_ref[...] = jnp.sum(pt * pt, axis=0, keepdims=True)
    paug_ref[...] = jnp.concatenate(
        [pt, jnp.zeros((5, NPAD), _F32)], axis=0).astype(jnp.bfloat16)


def _prep(feats, pts, ptsT_scaled, w1a, w1c):
    return pl.pallas_call(
        _prep_body,
        out_shape=[
            jax.ShapeDtypeStruct((NP, HID), _F32),
            jax.ShapeDtypeStruct((8, NPAD), jnp.bfloat16),
            jax.ShapeDtypeStruct((1, NPAD), _F32),
        ],
    )(feats, pts, ptsT_scaled, w1a, w1c)


def _knn_body(qbf_ref, q2_ref, paug_ref, p2_ref, nbr_ref):
    qp = jnp.dot(qbf_ref[...], paug_ref[...], preferred_element_type=_F32)
    s = (q2_ref[...] + p2_ref[...]) - 2.0 * qp
    iota_f = lax.broadcasted_iota(jnp.int32, s.shape, 1).astype(_F32)
    big = jnp.float32(3e38)
    h = NPAD // 2
    sA, sB = s[:, :h], s[:, h:]
    iA, iB = iota_f[:, :h], iota_f[:, h:]
    cmpB = sB < sA
    s1 = jnp.where(cmpB, sB, sA)
    i1 = jnp.where(cmpB, iB, iA)
    lv = jnp.where(cmpB, sA, sB)
    li = jnp.where(cmpB, iA, iB)
    cols = []
    for k in range(KNN):
        m = jnp.min(s1, axis=1, keepdims=True)
        hit = s1 == m
        idxf = jnp.min(jnp.where(hit, i1, big), axis=1, keepdims=True)
        cols.append(idxf)
        if k < KNN - 1:
            s1 = jnp.where(hit, lv, s1)
            i1 = jnp.where(hit, li, i1)
            lv = jnp.where(hit, big, lv)
    nbr_ref[...] = jnp.concatenate(cols, axis=1).astype(jnp.int32)


def _knn(qbf, q2, paug, p2row):
    return pl.pallas_call(
        _knn_body,
        grid=(NV // BQ,),
        in_specs=[
            pl.BlockSpec((BQ, 8), lambda i: (i, 0)),
            pl.BlockSpec((BQ, 1), lambda i: (i, 0)),
            pl.BlockSpec((8, NPAD), lambda i: (0, 0)),
            pl.BlockSpec((1, NPAD), lambda i: (0, 0)),
        ],
        out_specs=pl.BlockSpec((BQ, KNN), lambda i: (i, 0)),
        out_shape=jax.ShapeDtypeStruct((NV, KNN), jnp.int32),
    )(qbf, q2, paug, p2row)


def _sc_gather(p_table, nbr2d):
    info = plsc.get_sparse_core_info()
    nc, ns = info.num_cores, info.num_subcores
    nw = nc * ns
    rows = nbr2d.shape[0]
    rpw = rows // nw

    @functools.partial(
        pl.kernel,
        out_type=jax.ShapeDtypeStruct((NE, HID), _F32),
        mesh=plsc.VectorSubcoreMesh(core_axis_name="c", subcore_axis_name="s"),
        scratch_types=[
            pltpu.VMEM((rpw, 128), jnp.int32),
            pltpu.VMEM((128, HID), _F32),
            pltpu.SemaphoreType.DMA,
        ],
    )
    def gather_kernel(table_hbm, idx_hbm, out_hbm, idx_v, rows_v, sem):
        wid = lax.axis_index("s") * nc + lax.axis_index("c")
        base = wid * rpw
        pltpu.sync_copy(idx_hbm.at[pl.ds(base, rpw)], idx_v)

        def body(j, carry):
            pltpu.async_copy(table_hbm.at[idx_v.at[j]], rows_v, sem).wait()
            pltpu.sync_copy(rows_v, out_hbm.at[pl.ds((base + j) * 128, 128)])
            return carry

        lax.fori_loop(0, rpw, body, 0)

    return gather_kernel(p_table, nbr2d)


def _mlp_body(gath_ref, gfeat_ref, gneg_ref, w1b_ref, w1c_ref, b1_ref,
              w2_ref, b2_ref, w3_ref, b3_ref, w4_ref, b4_ref, out_ref):
    G = (
        jnp.dot(gfeat_ref[...], w1b_ref[...], preferred_element_type=_F32)
        + jnp.dot(gneg_ref[...], w1c_ref[...], preferred_element_type=_F32)
        + b1_ref[...]
    )
    hsum = jnp.zeros((VB, HID), _F32)
    for k in range(KNN):
        hsum = hsum + jax.nn.gelu(gath_ref[k] + G)
    hbar = hsum * (1.0 / KNN)
    red = jnp.dot(hbar, w2_ref[...], preferred_element_type=_F32) + b2_ref[...]
    h2 = jax.nn.gelu(
        jnp.dot(red, w3_ref[...], preferred_element_type=_F32) + b3_ref[...])
    out_ref[...] = (
        jnp.dot(h2, w4_ref[...], preferred_element_type=_F32) + b4_ref[...])


def _mlp(gath, gfeat, gneg, w1b, w1c, b1, w2, b2, w3, b3, w4, b4):
    full = lambda shape: pl.BlockSpec(shape, lambda i: (0, 0))
    return pl.pallas_call(
        _mlp_body,
        grid=(NV // VB,),
        in_specs=[
            pl.BlockSpec((KNN, VB, HID), lambda i: (0, i, 0)),
            pl.BlockSpec((VB, PCH), lambda i: (i, 0)),
            pl.BlockSpec((VB, 3), lambda i: (i, 0)),
            full((PCH, HID)), full((3, HID)), full((1, HID)),
            full((HID, 64)), full((1, 64)),
            full((64, HID)), full((1, HID)),
            full((HID, 64)), full((1, 64)),
        ],
        out_specs=pl.BlockSpec((VB, 64), lambda i: (i, 0)),
        out_shape=jax.ShapeDtypeStruct((NV, 64), _F32),
    )(gath, gfeat, gneg, w1b, w1c, b1, w2, b2, w3, b3, w4, b4)


def kernel(vertices, features, W1, b1, W2, b2, W3, b3, W4, b4):
    pts = vertices[0]
    feats = features[0]
    w1a, w1b, w1c = W1[:64], W1[64:160], W1[160:]

    pad = jnp.full((NPAD - NP, 3), 1e15, _F32)
    ptsT_scaled = jnp.concatenate([pts * SCALER, pad], axis=0).T

    p_table, paug, p2row = _prep(feats, pts, ptsT_scaled, w1a, w1c)
    qbf = jnp.asarray(_QBF_F32).astype(jnp.bfloat16)
    nbr = _knn(qbf, jnp.asarray(_Q2), paug, p2row)
    gath = _sc_gather(p_table, nbr.T.reshape(NE // 128, 128))
    out = _mlp(
        gath.reshape(KNN, NV, HID),
        jnp.asarray(_GRID_FEAT), jnp.asarray(-_GRID_FLAT),
        w1b, w1c, b1.reshape(1, HID),
        W2, b2.reshape(1, 64), W3, b3.reshape(1, HID), W4, b4.reshape(1, 64))
    return out.reshape(1, RES, RES, RES, 64)


# speedup vs baseline: 3.3498x; 1.0101x over previous
import functools

import jax
import jax.numpy as jnp
import numpy as np
from jax import lax
from jax.experimental import pallas as pl
from jax.experimental.pallas import tpu as pltpu
from jax.experimental.pallas import tpu_sc as plsc

RES = 32
NV = RES * RES * RES
KNN = 8
NP = 20000
NPAD = 20480
HID = 128
PE_DIM = 32
PCH = 3 * PE_DIM
SCALER = 32.0

BQ = 64
VB = 1024
NE = NV * KNN

_F32 = jnp.float32


def _host_grid_constants():
    ax = np.linspace(0.0, 1.0, RES, dtype=np.float32)
    g = np.stack(np.meshgrid(ax, ax, ax, indexing='ij'), axis=-1)
    grid_flat = g.reshape(-1, 3).astype(np.float32)
    freqs = (2.0 * np.pi) * (2.0 ** np.arange(PE_DIM // 2, dtype=np.float32))
    xf = grid_flat[..., None].astype(np.float32) * freqs
    enc = np.concatenate([np.sin(xf), np.cos(xf)], axis=-1)
    grid_feat = enc.reshape(NV, 3 * PE_DIM).astype(np.float32)
    q = grid_flat * SCALER
    qbf = np.zeros((NV, 8), dtype=np.float32)
    qbf[:, 0:3] = q
    q2 = (q * q).sum(axis=1, keepdims=True).astype(np.float32)
    return grid_flat, grid_feat, qbf, q2


_GRID_FLAT, _GRID_FEAT, _QBF_F32, _Q2 = _host_grid_constants()


def _prep_body(feats_ref, pts_ref, ptsT_ref, w1a_ref, w1c_ref,
               p_ref, paug_ref, p2_ref):
    p_ref[...] = (
        jnp.dot(feats_ref[...], w1a_ref[...], preferred_element_type=_F32)
        + jnp.dot(pts_ref[...], w1c_ref[...], preferred_element_type=_F32)
    )
    pt = ptsT_ref[...]
    p2_ref[...] = jnp.sum(pt * pt, axis=0, keepdims=True)
    paug_ref[...] = jnp.concatenate(
        [pt, jnp.zeros((5, NPAD), _F32)], axis=0).astype(jnp.bfloat16)


def _prep(feats, pts, ptsT_scaled, w1a, w1c):
    return pl.pallas_call(
        _prep_body,
        out_shape=[
            jax.ShapeDtypeStruct((NP, HID), _F32),
            jax.ShapeDtypeStruct((8, NPAD), jnp.bfloat16),
            jax.ShapeDtypeStruct((1, NPAD), _F32),
        ],
    )(feats, pts, ptsT_scaled, w1a, w1c)


def _knn_body(qbf_ref, q2_ref, paug_ref, p2_ref, nbr_ref):
    qp = jnp.dot(qbf_ref[...], paug_ref[...], preferred_element_type=_F32)
    s = (q2_ref[...] + p2_ref[...]) - 2.0 * qp
    iota_f = lax.broadcasted_iota(jnp.int32, s.shape, 1).astype(_F32)
    big = jnp.float32(3e38)
    h = NPAD // 2
    sA, sB = s[:, :h], s[:, h:]
    iA, iB = iota_f[:, :h], iota_f[:, h:]
    cmpB = sB < sA
    s1 = jnp.where(cmpB, sB, sA)
    i1 = jnp.where(cmpB, iB, iA)
    lv = jnp.where(cmpB, sA, sB)
    li = jnp.where(cmpB, iA, iB)
    cols = []
    for k in range(KNN):
        m = jnp.min(s1, axis=1, keepdims=True)
        hit = s1 == m
        idxf = jnp.min(jnp.where(hit, i1, big), axis=1, keepdims=True)
        cols.append(idxf)
        if k < KNN - 1:
            s1 = jnp.where(hit, lv, s1)
            i1 = jnp.where(hit, li, i1)
            lv = jnp.where(hit, big, lv)
    nbr_ref[...] = jnp.concatenate(cols, axis=1).astype(jnp.int32)


def _knn(qbf, q2, paug, p2row):
    return pl.pallas_call(
        _knn_body,
        grid=(NV // BQ,),
        in_specs=[
            pl.BlockSpec((BQ, 8), lambda i: (i, 0)),
            pl.BlockSpec((BQ, 1), lambda i: (i, 0)),
            pl.BlockSpec((8, NPAD), lambda i: (0, 0)),
            pl.BlockSpec((1, NPAD), lambda i: (0, 0)),
        ],
        out_specs=pl.BlockSpec((BQ, KNN), lambda i: (i, 0)),
        out_shape=jax.ShapeDtypeStruct((NV, KNN), jnp.int32),
    )(qbf, q2, paug, p2row)


def _sc_gather(p_table, nbr2d):
    info = plsc.get_sparse_core_info()
    nc, ns = info.num_cores, info.num_subcores
    nw = nc * ns
    rows = nbr2d.shape[0]
    rpw = rows // nw

    @functools.partial(
        pl.kernel,
        out_type=jax.ShapeDtypeStruct((NE, HID), _F32),
        mesh=plsc.VectorSubcoreMesh(core_axis_name="c", subcore_axis_name="s"),
        scratch_types=[
            pltpu.VMEM((rpw, 128), jnp.int32),
            pltpu.VMEM((128, HID), _F32),
            pltpu.VMEM((128, HID), _F32),
            pltpu.SemaphoreType.DMA,
            pltpu.SemaphoreType.DMA,
        ],
    )
    def gather_kernel(table_hbm, idx_hbm, out_hbm, idx_v, rows0, rows1, s0, s1):
        wid = lax.axis_index("s") * nc + lax.axis_index("c")
        base = wid * rpw
        pltpu.sync_copy(idx_hbm.at[pl.ds(base, rpw)], idx_v)

        def body(t, carry):
            j0 = 2 * t
            c0 = pltpu.async_copy(table_hbm.at[idx_v.at[j0]], rows0, s0)
            c1 = pltpu.async_copy(table_hbm.at[idx_v.at[j0 + 1]], rows1, s1)
            c0.wait()
            pltpu.sync_copy(rows0, out_hbm.at[pl.ds((base + j0) * 128, 128)])
            c1.wait()
            pltpu.sync_copy(rows1, out_hbm.at[pl.ds((base + j0 + 1) * 128, 128)])
            return carry

        lax.fori_loop(0, rpw // 2, body, 0)

    return gather_kernel(p_table, nbr2d)


def _mlp_body(gath_ref, gfeat_ref, gneg_ref, w1b_ref, w1c_ref, b1_ref,
              w2_ref, b2_ref, w3_ref, b3_ref, w4_ref, b4_ref, out_ref):
    G = (
        jnp.dot(gfeat_ref[...], w1b_ref[...], preferred_element_type=_F32)
        + jnp.dot(gneg_ref[...], w1c_ref[...], preferred_element_type=_F32)
        + b1_ref[...]
    )
    hsum = jnp.zeros((VB, HID), _F32)
    for k in range(KNN):
        hsum = hsum + jax.nn.gelu(gath_ref[k] + G)
    hbar = hsum * (1.0 / KNN)
    red = jnp.dot(hbar, w2_ref[...], preferred_element_type=_F32) + b2_ref[...]
    h2 = jax.nn.gelu(
        jnp.dot(red, w3_ref[...], preferred_element_type=_F32) + b3_ref[...])
    out_ref[...] = (
        jnp.dot(h2, w4_ref[...], preferred_element_type=_F32) + b4_ref[...])


def _mlp(gath, gfeat, gneg, w1b, w1c, b1, w2, b2, w3, b3, w4, b4):
    full = lambda shape: pl.BlockSpec(shape, lambda i: (0, 0))
    return pl.pallas_call(
        _mlp_body,
        grid=(NV // VB,),
        in_specs=[
            pl.BlockSpec((KNN, VB, HID), lambda i: (0, i, 0)),
            pl.BlockSpec((VB, PCH), lambda i: (i, 0)),
            pl.BlockSpec((VB, 3), lambda i: (i, 0)),
            full((PCH, HID)), full((3, HID)), full((1, HID)),
            full((HID, 64)), full((1, 64)),
            full((64, HID)), full((1, HID)),
            full((HID, 64)), full((1, 64)),
        ],
        out_specs=pl.BlockSpec((VB, 64), lambda i: (i, 0)),
        out_shape=jax.ShapeDtypeStruct((NV, 64), _F32),
    )(gath, gfeat, gneg, w1b, w1c, b1, w2, b2, w3, b3, w4, b4)


def kernel(vertices, features, W1, b1, W2, b2, W3, b3, W4, b4):
    pts = vertices[0]
    feats = features[0]
    w1a, w1b, w1c = W1[:64], W1[64:160], W1[160:]

    pad = jnp.full((NPAD - NP, 3), 1e15, _F32)
    ptsT_scaled = jnp.concatenate([pts * SCALER, pad], axis=0).T

    p_table, paug, p2row = _prep(feats, pts, ptsT_scaled, w1a, w1c)
    qbf = jnp.asarray(_QBF_F32).astype(jnp.bfloat16)
    nbr = _knn(qbf, jnp.asarray(_Q2), paug, p2row)
    gath = _sc_gather(p_table, nbr.T.reshape(NE // 128, 128))
    out = _mlp(
        gath.reshape(KNN, NV, HID),
        jnp.asarray(_GRID_FEAT), jnp.asarray(-_GRID_FLAT),
        w1b, w1c, b1.reshape(1, HID),
        W2, b2.reshape(1, 64), W3, b3.reshape(1, HID), W4, b4.reshape(1, 64))
    return out.reshape(1, RES, RES, RES, 64)


# speedup vs baseline: 3.4435x; 1.0280x over previous
import functools

import jax
import jax.numpy as jnp
import numpy as np
from jax import lax
from jax.experimental import pallas as pl
from jax.experimental.pallas import tpu as pltpu
from jax.experimental.pallas import tpu_sc as plsc

RES = 32
NV = RES * RES * RES
KNN = 8
NP = 20000
NPAD = 20480
HID = 128
PE_DIM = 32
PCH = 3 * PE_DIM
SCALER = 32.0

BQ = 128
VB = 1024
NE = NV * KNN

_F32 = jnp.float32


def _host_grid_constants():
    ax = np.linspace(0.0, 1.0, RES, dtype=np.float32)
    g = np.stack(np.meshgrid(ax, ax, ax, indexing='ij'), axis=-1)
    grid_flat = g.reshape(-1, 3).astype(np.float32)
    freqs = (2.0 * np.pi) * (2.0 ** np.arange(PE_DIM // 2, dtype=np.float32))
    xf = grid_flat[..., None].astype(np.float32) * freqs
    enc = np.concatenate([np.sin(xf), np.cos(xf)], axis=-1)
    grid_feat = enc.reshape(NV, 3 * PE_DIM).astype(np.float32)
    q = grid_flat * SCALER
    qbf = np.zeros((NV, 8), dtype=np.float32)
    qbf[:, 0:3] = q
    q2 = (q * q).sum(axis=1, keepdims=True).astype(np.float32)
    return grid_flat, grid_feat, qbf, q2


_GRID_FLAT, _GRID_FEAT, _QBF_F32, _Q2 = _host_grid_constants()


def _prep_body(feats_ref, pts_ref, ptsT_ref, w1a_ref, w1c_ref,
               p_ref, paug_ref, p2_ref):
    p_ref[...] = (
        jnp.dot(feats_ref[...], w1a_ref[...], preferred_element_type=_F32)
        + jnp.dot(pts_ref[...], w1c_ref[...], preferred_element_type=_F32)
    )
    pt = ptsT_ref[...]
    p2_ref[...] = jnp.sum(pt * pt, axis=0, keepdims=True)
    paug_ref[...] = jnp.concatenate(
        [pt, jnp.zeros((5, NPAD), _F32)], axis=0).astype(jnp.bfloat16)


def _prep(feats, pts, ptsT_scaled, w1a, w1c):
    return pl.pallas_call(
        _prep_body,
        out_shape=[
            jax.ShapeDtypeStruct((NP, HID), _F32),
            jax.ShapeDtypeStruct((8, NPAD), jnp.bfloat16),
            jax.ShapeDtypeStruct((1, NPAD), _F32),
        ],
    )(feats, pts, ptsT_scaled, w1a, w1c)


def _knn_body(qbf_ref, q2_ref, paug_ref, p2_ref, nbr_ref):
    qp = jnp.dot(qbf_ref[...], paug_ref[...], preferred_element_type=_F32)
    s = (q2_ref[...] + p2_ref[...]) - 2.0 * qp
    big = jnp.float32(3e38)
    h = NPAD // 2
    sA, sB = s[:, :h], s[:, h:]
    iA = lax.broadcasted_iota(jnp.int32, sA.shape, 1).astype(_F32)
    iB = iA + jnp.float32(h)
    cmpB = sB < sA
    s1 = jnp.where(cmpB, sB, sA)
    i1 = jnp.where(cmpB, iB, iA)
    lv = jnp.where(cmpB, sA, sB)
    li = jnp.where(cmpB, iA, iB)
    cols = []
    for k in range(KNN):
        m = jnp.min(s1, axis=1, keepdims=True)
        hit = s1 == m
        idxf = jnp.min(jnp.where(hit, i1, big), axis=1, keepdims=True)
        cols.append(idxf)
        if k < KNN - 1:
            s1 = jnp.where(hit, lv, s1)
            i1 = jnp.where(hit, li, i1)
            lv = jnp.where(hit, big, lv)
    nbr_ref[...] = jnp.concatenate(cols, axis=1).astype(jnp.int32)


def _knn(qbf, q2, paug, p2row):
    return pl.pallas_call(
        _knn_body,
        grid=(NV // BQ,),
        in_specs=[
            pl.BlockSpec((BQ, 8), lambda i: (i, 0)),
            pl.BlockSpec((BQ, 1), lambda i: (i, 0)),
            pl.BlockSpec((8, NPAD), lambda i: (0, 0)),
            pl.BlockSpec((1, NPAD), lambda i: (0, 0)),
        ],
        out_specs=pl.BlockSpec((BQ, KNN), lambda i: (i, 0)),
        out_shape=jax.ShapeDtypeStruct((NV, KNN), jnp.int32),
    )(qbf, q2, paug, p2row)


def _sc_gather(p_table, nbr2d):
    info = plsc.get_sparse_core_info()
    nc, ns = info.num_cores, info.num_subcores
    nw = nc * ns
    rows = nbr2d.shape[0]
    rpw = rows // nw

    @functools.partial(
        pl.kernel,
        out_type=jax.ShapeDtypeStruct((NE, HID), _F32),
        mesh=plsc.VectorSubcoreMesh(core_axis_name="c", subcore_axis_name="s"),
        scratch_types=[
            pltpu.VMEM((rpw, 128), jnp.int32),
            pltpu.VMEM((128, HID), _F32),
            pltpu.VMEM((128, HID), _F32),
            pltpu.SemaphoreType.DMA,
            pltpu.SemaphoreType.DMA,
        ],
    )
    def gather_kernel(table_hbm, idx_hbm, out_hbm, idx_v, rows0, rows1, s0, s1):
        wid = lax.axis_index("s") * nc + lax.axis_index("c")
        base = wid * rpw
        pltpu.sync_copy(idx_hbm.at[pl.ds(base, rpw)], idx_v)

        def body(t, carry):
            j0 = 2 * t
            c0 = pltpu.async_copy(table_hbm.at[idx_v.at[j0]], rows0, s0)
            c1 = pltpu.async_copy(table_hbm.at[idx_v.at[j0 + 1]], rows1, s1)
            c0.wait()
            pltpu.sync_copy(rows0, out_hbm.at[pl.ds((base + j0) * 128, 128)])
            c1.wait()
            pltpu.sync_copy(rows1, out_hbm.at[pl.ds((base + j0 + 1) * 128, 128)])
            return carry

        lax.fori_loop(0, rpw // 2, body, 0)

    return gather_kernel(p_table, nbr2d)


def _mlp_body(gath_ref, gfeat_ref, gneg_ref, w1b_ref, w1c_ref, b1_ref,
              w2_ref, b2_ref, w3_ref, b3_ref, w4_ref, b4_ref, out_ref):
    G = (
        jnp.dot(gfeat_ref[...], w1b_ref[...], preferred_element_type=_F32)
        + jnp.dot(gneg_ref[...], w1c_ref[...], preferred_element_type=_F32)
        + b1_ref[...]
    )
    hsum = jnp.zeros((VB, HID), _F32)
    for k in range(KNN):
        hsum = hsum + jax.nn.gelu(gath_ref[k] + G)
    hbar = hsum * (1.0 / KNN)
    red = jnp.dot(hbar, w2_ref[...], preferred_element_type=_F32) + b2_ref[...]
    h2 = jax.nn.gelu(
        jnp.dot(red, w3_ref[...], preferred_element_type=_F32) + b3_ref[...])
    out_ref[...] = (
        jnp.dot(h2, w4_ref[...], preferred_element_type=_F32) + b4_ref[...])


def _mlp(gath, gfeat, gneg, w1b, w1c, b1, w2, b2, w3, b3, w4, b4):
    full = lambda shape: pl.BlockSpec(shape, lambda i: (0, 0))
    return pl.pallas_call(
        _mlp_body,
        grid=(NV // VB,),
        in_specs=[
            pl.BlockSpec((KNN, VB, HID), lambda i: (0, i, 0)),
            pl.BlockSpec((VB, PCH), lambda i: (i, 0)),
            pl.BlockSpec((VB, 3), lambda i: (i, 0)),
            full((PCH, HID)), full((3, HID)), full((1, HID)),
            full((HID, 64)), full((1, 64)),
            full((64, HID)), full((1, HID)),
            full((HID, 64)), full((1, 64)),
        ],
        out_specs=pl.BlockSpec((VB, 64), lambda i: (i, 0)),
        out_shape=jax.ShapeDtypeStruct((NV, 64), _F32),
    )(gath, gfeat, gneg, w1b, w1c, b1, w2, b2, w3, b3, w4, b4)


def kernel(vertices, features, W1, b1, W2, b2, W3, b3, W4, b4):
    pts = vertices[0]
    feats = features[0]
    w1a, w1b, w1c = W1[:64], W1[64:160], W1[160:]

    pad = jnp.full((NPAD - NP, 3), 1e15, _F32)
    ptsT_scaled = jnp.concatenate([pts * SCALER, pad], axis=0).T

    p_table, paug, p2row = _prep(feats, pts, ptsT_scaled, w1a, w1c)
    qbf = jnp.asarray(_QBF_F32).astype(jnp.bfloat16)
    nbr = _knn(qbf, jnp.asarray(_Q2), paug, p2row)
    gath = _sc_gather(p_table, nbr.T.reshape(NE // 128, 128))
    out = _mlp(
        gath.reshape(KNN, NV, HID),
        jnp.asarray(_GRID_FEAT), jnp.asarray(-_GRID_FLAT),
        w1b, w1c, b1.reshape(1, HID),
        W2, b2.reshape(1, 64), W3, b3.reshape(1, HID), W4, b4.reshape(1, 64))
    return out.reshape(1, RES, RES, RES, 64)


# speedup vs baseline: 3.4507x; 1.0021x over previous
import functools

import jax
import jax.numpy as jnp
import numpy as np
from jax import lax
from jax.experimental import pallas as pl
from jax.experimental.pallas import tpu as pltpu
from jax.experimental.pallas import tpu_sc as plsc

RES = 32
NV = RES * RES * RES
KNN = 8
NP = 20000
NPAD = 20480
HID = 128
PE_DIM = 32
PCH = 3 * PE_DIM
SCALER = 32.0

BQ = 128
VB = 1024
NE = NV * KNN

_F32 = jnp.float32


def _host_grid_constants():
    ax = np.linspace(0.0, 1.0, RES, dtype=np.float32)
    g = np.stack(np.meshgrid(ax, ax, ax, indexing='ij'), axis=-1)
    grid_flat = g.reshape(-1, 3).astype(np.float32)
    freqs = (2.0 * np.pi) * (2.0 ** np.arange(PE_DIM // 2, dtype=np.float32))
    xf = grid_flat[..., None].astype(np.float32) * freqs
    enc = np.concatenate([np.sin(xf), np.cos(xf)], axis=-1)
    grid_feat = enc.reshape(NV, 3 * PE_DIM).astype(np.float32)
    q = grid_flat * SCALER
    qbf = np.zeros((NV, 8), dtype=np.float32)
    qbf[:, 0:3] = q
    q2 = (q * q).sum(axis=1, keepdims=True).astype(np.float32)
    return grid_flat, grid_feat, qbf, q2


_GRID_FLAT, _GRID_FEAT, _QBF_F32, _Q2 = _host_grid_constants()


def _prep_body(feats_ref, pts_ref, ptsT_ref, w1a_ref, w1c_ref,
               p_ref, paug_ref, p2_ref):
    p_ref[...] = (
        jnp.dot(feats_ref[...], w1a_ref[...], preferred_element_type=_F32)
        + jnp.dot(pts_ref[...], w1c_ref[...], preferred_element_type=_F32)
    )
    pt = ptsT_ref[...]
    p2_ref[...] = jnp.sum(pt * pt, axis=0, keepdims=True)
    paug_ref[...] = jnp.concatenate(
        [pt, jnp.zeros((5, NPAD), _F32)], axis=0).astype(jnp.bfloat16)


def _prep(feats, pts, ptsT_scaled, w1a, w1c):
    return pl.pallas_call(
        _prep_body,
        out_shape=[
            jax.ShapeDtypeStruct((NP, HID), _F32),
            jax.ShapeDtypeStruct((8, NPAD), jnp.bfloat16),
            jax.ShapeDtypeStruct((1, NPAD), _F32),
        ],
    )(feats, pts, ptsT_scaled, w1a, w1c)


def _knn_body(qbf_ref, q2_ref, paug_ref, p2_ref, nbr_ref):
    qp = jnp.dot(qbf_ref[...], paug_ref[...], preferred_element_type=_F32)
    s = (q2_ref[...] + p2_ref[...]) - 2.0 * qp
    big = jnp.float32(3e38)
    h = NPAD // 2
    sA, sB = s[:, :h], s[:, h:]
    iA = lax.broadcasted_iota(jnp.int32, sA.shape, 1).astype(_F32)
    iB = iA + jnp.float32(h)
    cmpB = sB < sA
    s1 = jnp.where(cmpB, sB, sA)
    i1 = jnp.where(cmpB, iB, iA)
    lv = jnp.where(cmpB, sA, sB)
    li = jnp.where(cmpB, iA, iB)
    cols = []
    for k in range(KNN):
        m = jnp.min(s1, axis=1, keepdims=True)
        hit = s1 == m
        idxf = jnp.min(jnp.where(hit, i1, big), axis=1, keepdims=True)
        cols.append(idxf)
        if k < KNN - 1:
            s1 = jnp.where(hit, lv, s1)
            i1 = jnp.where(hit, li, i1)
            lv = jnp.where(hit, big, lv)
    nbr_ref[...] = jnp.concatenate(cols, axis=1).astype(jnp.int32)


def _knn(qbf, q2, paug, p2row):
    return pl.pallas_call(
        _knn_body,
        grid=(NV // BQ,),
        in_specs=[
            pl.BlockSpec((BQ, 8), lambda i: (i, 0)),
            pl.BlockSpec((BQ, 1), lambda i: (i, 0)),
            pl.BlockSpec((8, NPAD), lambda i: (0, 0)),
            pl.BlockSpec((1, NPAD), lambda i: (0, 0)),
        ],
        out_specs=pl.BlockSpec((BQ, KNN), lambda i: (i, 0)),
        out_shape=jax.ShapeDtypeStruct((NV, KNN), jnp.int32),
    )(qbf, q2, paug, p2row)


def _sc_gather(p_table, nbr2d):
    info = plsc.get_sparse_core_info()
    nc, ns = info.num_cores, info.num_subcores
    nw = nc * ns
    rows = nbr2d.shape[0]
    rpw = rows // nw

    @functools.partial(
        pl.kernel,
        out_type=jax.ShapeDtypeStruct((NE, HID), _F32),
        mesh=plsc.VectorSubcoreMesh(core_axis_name="c", subcore_axis_name="s"),
        scratch_types=[
            pltpu.VMEM((rpw, 128), jnp.int32),
            pltpu.VMEM((128, HID), _F32),
            pltpu.VMEM((128, HID), _F32),
            pltpu.VMEM((128, HID), _F32),
            pltpu.VMEM((128, HID), _F32),
            pltpu.SemaphoreType.DMA,
            pltpu.SemaphoreType.DMA,
            pltpu.SemaphoreType.DMA,
            pltpu.SemaphoreType.DMA,
        ],
    )
    def gather_kernel(table_hbm, idx_hbm, out_hbm, idx_v,
                      rows0, rows1, rows2, rows3, s0, s1, s2, s3):
        wid = lax.axis_index("s") * nc + lax.axis_index("c")
        base = wid * rpw
        pltpu.sync_copy(idx_hbm.at[pl.ds(base, rpw)], idx_v)
        bufs = (rows0, rows1, rows2, rows3)
        sems = (s0, s1, s2, s3)

        def body(t, carry):
            j0 = 4 * t
            cps = [pltpu.async_copy(table_hbm.at[idx_v.at[j0 + b]],
                                    bufs[b], sems[b]) for b in range(4)]
            for b in range(4):
                cps[b].wait()
                pltpu.sync_copy(bufs[b],
                                out_hbm.at[pl.ds((base + j0 + b) * 128, 128)])
            return carry

        lax.fori_loop(0, rpw // 4, body, 0)

    return gather_kernel(p_table, nbr2d)


def _mlp_body(gath_ref, gfeat_ref, gneg_ref, w1b_ref, w1c_ref, b1_ref,
              w2_ref, b2_ref, w3_ref, b3_ref, w4_ref, b4_ref, out_ref):
    G = (
        jnp.dot(gfeat_ref[...], w1b_ref[...], preferred_element_type=_F32)
        + jnp.dot(gneg_ref[...], w1c_ref[...], preferred_element_type=_F32)
        + b1_ref[...]
    )
    hsum = jnp.zeros((VB, HID), _F32)
    for k in range(KNN):
        hsum = hsum + jax.nn.gelu(gath_ref[k] + G)
    hbar = hsum * (1.0 / KNN)
    red = jnp.dot(hbar, w2_ref[...], preferred_element_type=_F32) + b2_ref[...]
    h2 = jax.nn.gelu(
        jnp.dot(red, w3_ref[...], preferred_element_type=_F32) + b3_ref[...])
    out_ref[...] = (
        jnp.dot(h2, w4_ref[...], preferred_element_type=_F32) + b4_ref[...])


def _mlp(gath, gfeat, gneg, w1b, w1c, b1, w2, b2, w3, b3, w4, b4):
    full = lambda shape: pl.BlockSpec(shape, lambda i: (0, 0))
    return pl.pallas_call(
        _mlp_body,
        grid=(NV // VB,),
        in_specs=[
            pl.BlockSpec((KNN, VB, HID), lambda i: (0, i, 0)),
            pl.BlockSpec((VB, PCH), lambda i: (i, 0)),
            pl.BlockSpec((VB, 3), lambda i: (i, 0)),
            full((PCH, HID)), full((3, HID)), full((1, HID)),
            full((HID, 64)), full((1, 64)),
            full((64, HID)), full((1, HID)),
            full((HID, 64)), full((1, 64)),
        ],
        out_specs=pl.BlockSpec((VB, 64), lambda i: (i, 0)),
        out_shape=jax.ShapeDtypeStruct((NV, 64), _F32),
    )(gath, gfeat, gneg, w1b, w1c, b1, w2, b2, w3, b3, w4, b4)


def kernel(vertices, features, W1, b1, W2, b2, W3, b3, W4, b4):
    pts = vertices[0]
    feats = features[0]
    w1a, w1b, w1c = W1[:64], W1[64:160], W1[160:]

    pad = jnp.full((NPAD - NP, 3), 1e15, _F32)
    ptsT_scaled = jnp.concatenate([pts * SCALER, pad], axis=0).T

    p_table, paug, p2row = _prep(feats, pts, ptsT_scaled, w1a, w1c)
    qbf = jnp.asarray(_QBF_F32).astype(jnp.bfloat16)
    nbr = _knn(qbf, jnp.asarray(_Q2), paug, p2row)
    gath = _sc_gather(p_table, nbr.T.reshape(NE // 128, 128))
    out = _mlp(
        gath.reshape(KNN, NV, HID),
        jnp.asarray(_GRID_FEAT), jnp.asarray(-_GRID_FLAT),
        w1b, w1c, b1.reshape(1, HID),
        W2, b2.reshape(1, 64), W3, b3.reshape(1, HID), W4, b4.reshape(1, 64))
    return out.reshape(1, RES, RES, RES, 64)
